# Initial kernel scaffold; baseline (speedup 1.0000x reference)
import jax
import jax.numpy as jnp
from jax.experimental import pallas as pl


def kernel(x, edge_index, W1, W2, epsilons, mlp_w1, mlp_b1, mlp_w2, mlp_b2):
    raise NotImplementedError("write your pallas kernel here")


# speedup vs baseline: 228.7366x; 228.7366x over previous
import functools

import jax
import jax.numpy as jnp
from jax import lax
from jax.experimental import pallas as pl
from jax.experimental.pallas import tpu as pltpu
from jax.experimental.pallas import tpu_sc as plsc

D = 2
FD = 64
DF = D * FD
NCORE = 2
NSUB = 16
NW = NCORE * NSUB
C = 80


def _node_pad(n):
    per_tile = ((n + NSUB * C - 1) // (NSUB * C)) * C
    return NSUB * per_tile, per_tile


def _deg_sc(e_total, n_pad):
    ept = e_total // NW
    nch = ept // C
    mesh = plsc.VectorSubcoreMesh(core_axis_name="c", subcore_axis_name="s")

    @functools.partial(
        pl.kernel,
        mesh=mesh,
        out_type=jax.ShapeDtypeStruct((NW, n_pad * 8), jnp.float32),
        scratch_types=[
            pltpu.VMEM((C,), jnp.int32),
            pltpu.VMEM((n_pad * 8,), jnp.float32),
        ],
    )
    def deg_k(row_hbm, out_hbm, idx_v, hist_l):
        cid = lax.axis_index("c")
        sid = lax.axis_index("s")
        wid = cid * NSUB + sid
        zero16 = jnp.zeros((16,), jnp.float32)
        lanes = lax.iota(jnp.int32, 16)
        e0 = jnp.where(lanes == 0, jnp.full((16,), 1.0, jnp.float32), zero16)

        def z(i, _):
            hist_l[pl.ds(i * 16, 16)] = zero16
            return 0

        lax.fori_loop(0, n_pad * 8 // 16, z, 0)

        base = wid * ept

        def chunk(i, _):
            pltpu.sync_copy(row_hbm.at[pl.ds(base + i * C, C)], idx_v)

            def group(g, _):
                idx16 = idx_v[pl.ds(g * 16, 16)]
                off16 = idx16 * 8
                for l in range(16):
                    off = off16[l]
                    hist_l[pl.ds(off, 16)] = hist_l[pl.ds(off, 16)] + e0
                return 0

            lax.fori_loop(0, C // 16, group, 0)
            return 0

        lax.fori_loop(0, nch, chunk, 0)
        pltpu.sync_copy(hist_l, out_hbm.at[wid])

    return deg_k


def _edge_sc(e_total, n_pad, rows_per_tile):
    ept = e_total // NW
    nch = ept // C
    zch = rows_per_tile // C
    mesh = plsc.VectorSubcoreMesh(core_axis_name="c", subcore_axis_name="s")

    @functools.partial(
        pl.kernel,
        mesh=mesh,
        out_type=jax.ShapeDtypeStruct((NCORE, n_pad, DF), jnp.float32),
        scratch_types=[
            pltpu.VMEM((C,), jnp.int32),
            pltpu.VMEM((C,), jnp.int32),
            pltpu.VMEM((C, DF), jnp.float32),
            pltpu.VMEM((C, DF), jnp.float32),
            pltpu.VMEM((C, DF), jnp.float32),
            pltpu.VMEM((C, DF), jnp.float32),
            pltpu.VMEM((4, FD), jnp.float32),
            pltpu.VMEM_SHARED((n_pad, DF), jnp.float32),
            pltpu.SemaphoreType.DMA,
            pltpu.SemaphoreType.DMA,
            pltpu.SemaphoreType.DMA,
        ],
    )
    def edge_k(row_hbm, col_hbm, uv_hbm, xs_hbm, consts_hbm, out_hbm,
               ridx, cidx, uvc, uvr, xsr, msg, cvm, agg_sh, sem1, sem2, sem3):
        cid = lax.axis_index("c")
        sid = lax.axis_index("s")
        wid = cid * NSUB + sid

        def zstore(i, _):
            msg[i // 8, pl.ds((i % 8) * 16, 16)] = jnp.zeros((16,), jnp.float32)
            return 0

        lax.fori_loop(0, C * 8, zstore, 0)

        def zcopy(i, _):
            pltpu.sync_copy(msg,
                            agg_sh.at[pl.ds(sid * rows_per_tile + i * C, C)])
            return 0

        lax.fori_loop(0, zch, zcopy, 0)
        pltpu.sync_copy(consts_hbm, cvm)
        plsc.subcore_barrier()

        b1v = [cvm[0, pl.ds(k * 16, 16)] for k in range(4)]
        w2v = [cvm[1, pl.ds(k * 16, 16)] for k in range(4)]
        b2e0 = cvm[2, pl.ds(0, 16)]

        base = wid * ept

        def chunk(ci, _):
            off = base + ci * C
            pltpu.sync_copy(row_hbm.at[pl.ds(off, C)], ridx)
            pltpu.sync_copy(col_hbm.at[pl.ds(off, C)], cidx)
            cp1 = pltpu.async_copy(uv_hbm.at[cidx], uvc, sem1)
            cp2 = pltpu.async_copy(uv_hbm.at[ridx], uvr, sem2)
            cp3 = pltpu.async_copy(xs_hbm.at[ridx], xsr, sem3)
            cp1.wait()
            cp2.wait()
            cp3.wait()

            def edge(e, _):
                acc = b2e0
                for k in range(4):
                    uk = uvc[e, pl.ds(k * 16, 16)]
                    vk = uvr[e, pl.ds(FD + k * 16, 16)]
                    hk = jnp.maximum(uk + vk + b1v[k], 0.0)
                    acc = acc + hk * w2v[k]
                lanes = lax.iota(jnp.int32, 16)
                dnums = lax.GatherDimensionNumbers(
                    offset_dims=(), collapsed_slice_dims=(0,),
                    start_index_map=(0,))
                for sh in (1, 2, 4, 8):
                    idx = jnp.bitwise_and(lanes + sh, 15)
                    rot = lax.gather(
                        acc, idx[:, None], dnums, (1,),
                        mode=lax.GatherScatterMode.PROMISE_IN_BOUNDS)
                    acc = acc + rot
                pv = acc
                s2 = 1.0 + pv * pv
                ib = lax.bitcast_convert_type(s2, jnp.int32)
                ib = jnp.int32(0x5F3759DF) - lax.shift_right_arithmetic(ib, 1)
                y = lax.bitcast_convert_type(ib, jnp.float32)
                for _ in range(3):
                    y = y * (1.5 - 0.5 * s2 * y * y)
                c_ = y
                t_ = pv * y
                q00 = jnp.where(pv == 0.0, c_, -c_)
                for k in range(4):
                    x0 = xsr[e, pl.ds(k * 16, 16)]
                    x1 = xsr[e, pl.ds(FD + k * 16, 16)]
                    msg[e, pl.ds(k * 16, 16)] = q00 * x0 - t_ * x1
                    msg[e, pl.ds(FD + k * 16, 16)] = c_ * x1 - t_ * x0
                return 0

            lax.fori_loop(0, C, edge, 0)
            pltpu.sync_copy(msg, agg_sh.at[cidx], add=True)
            return 0

        lax.fori_loop(0, nch, chunk, 0)
        plsc.subcore_barrier()

        def wb(i, _):
            r0 = sid * rows_per_tile + i * C
            pltpu.sync_copy(agg_sh.at[pl.ds(r0, C)],
                            out_hbm.at[cid, pl.ds(r0, C)])
            return 0

        lax.fori_loop(0, zch, wb, 0)

    return edge_k


def _node_tc(n):
    def body(x_ref, w_ref, deg_ref, cvec_ref, uv_ref, xs_ref, self_ref):
        xv = x_ref[...]
        r = jnp.dot(xv, w_ref[...], preferred_element_type=jnp.float32)
        dis = lax.rsqrt(deg_ref[:n, :])
        uv = r[:, :DF]
        xs = r[:, DF:] * dis
        uv_ref[...] = uv
        xs_ref[...] = xs
        b1 = cvec_ref[0:1, :FD]
        w2 = cvec_ref[1:2, :FD]
        h = jnp.maximum(uv[:, :FD] + uv[:, FD:] + b1, 0.0)
        p = jnp.sum(h * w2, axis=1, keepdims=True) + cvec_ref[2:3, 0:1]
        y = lax.rsqrt(1.0 + p * p)
        t = p * y
        q00 = jnp.where(p == 0.0, y, -y)
        x0 = xs[:, :FD]
        x1 = xs[:, FD:]
        self_ref[...] = jnp.concatenate(
            [q00 * x0 - t * x1, y * x1 - t * x0], axis=1)

    return body


def _final_tc(n):
    def body(x_ref, agg_ref, self_ref, deg_ref, coeff_ref, out_ref):
        a = agg_ref[0, :n, :] + agg_ref[1, :n, :] + self_ref[...]
        dis = lax.rsqrt(deg_ref[:n, :])
        z = a * dis
        elz = jnp.where(z > 0.0, z, jnp.exp(jnp.minimum(z, 0.0)) - 1.0)
        out_ref[...] = x_ref[...] - coeff_ref[0:1, :] * elz

    return body


def kernel(x, edge_index, W1, W2, epsilons, mlp_w1, mlp_b1, mlp_w2, mlp_b2):
    n = x.shape[0]
    e = edge_index.shape[1]
    n_pad, rows_per_tile = _node_pad(n)

    M = jnp.kron(W1, W2)
    WU = M @ mlp_w1[:, :DF].T
    WV = M @ mlp_w1[:, DF:].T
    Wcat = jnp.concatenate([WU, WV, M], axis=1)
    coeff = 1.0 + jnp.tanh(epsilons)
    coeff_row = jnp.tile(jnp.repeat(coeff, FD)[None, :], (8, 1))
    consts = jnp.stack([
        mlp_b1,
        mlp_w2[0],
        jnp.zeros((FD,), jnp.float32).at[0].set(mlp_b2[0]),
        jnp.zeros((FD,), jnp.float32),
    ])
    cvec = jnp.pad(consts, ((0, 4), (0, 0)))

    row = edge_index[0].astype(jnp.int32)
    col = edge_index[1].astype(jnp.int32)

    hists = _deg_sc(e, n_pad)(row)
    deg_col = (jnp.sum(hists.reshape(NW, n_pad, 8)[:, :, 0], axis=0) + 1.0).reshape(n_pad, 1)

    uv_tab, xs_tab, self_msg = pl.pallas_call(
        _node_tc(n),
        out_shape=[
            jax.ShapeDtypeStruct((n, DF), jnp.float32),
            jax.ShapeDtypeStruct((n, DF), jnp.float32),
            jax.ShapeDtypeStruct((n, DF), jnp.float32),
        ],
    )(x, Wcat, deg_col, cvec)

    agg = _edge_sc(e, n_pad, rows_per_tile)(row, col, uv_tab, xs_tab, consts)

    out = pl.pallas_call(
        _final_tc(n),
        out_shape=jax.ShapeDtypeStruct((n, DF), jnp.float32),
    )(x, agg, self_msg, deg_col, coeff_row)
    return out
